# baseline (speedup 1.0000x reference)
import jax
import jax.numpy as jnp
from jax import lax
from jax.experimental import pallas as pl
from jax.experimental.pallas import tpu as pltpu
from jax.experimental.pallas import tpu_sc as plsc

N_SRC = 100000
D = 128
NSEC = 100
NSEC_PAD = 128
B = 16384
SLICE_RANGE = 1000

NC = 2
NS = 16
NW = NC * NS

CHUNK = 400
SUB = 100
NSUB = CHUNK // SUB
NCHUNKS = N_SRC // CHUNK

N_SC = 64000
NCHUNKS_SC = N_SC // CHUNK
NCH_PER_W = NCHUNKS_SC // NW

TSEG = 2000
TSEG_GRID = (N_SRC - N_SC) // TSEG
TSEG_OFF = N_SC // TSEG

HBLK = 12544
HGRID = -(-N_SRC // HBLK)
N_SRC_PAD = HGRID * HBLK

TBLK = 2048
GRID = B // TBLK


def _seg_sum_body(src_emb, sectors, zeros_acc, out_sums,
                  rows_v0, rows_v1, idx_v0, idx_v1, acc_sh,
                  lsem0, lsem1, isem0, isem1):
    c = lax.axis_index("c")
    s = lax.axis_index("s")
    wid = c * NS + s

    @pl.when(s == 0)
    def _():
        pltpu.sync_copy(zeros_acc, acc_sh)

    plsc.subcore_barrier()

    rows = (rows_v0, rows_v1)
    idxs = (idx_v0, idx_v1)
    lsems = (lsem0, lsem1)
    isems = (isem0, isem1)

    def start(k):
        ci = k * NW + wid
        b = k % 2
        pltpu.async_copy(src_emb.at[pl.ds(ci * CHUNK, CHUNK)], rows[b],
                         lsems[b])
        pltpu.async_copy(sectors.at[ci], idxs[b], isems[b])

    def consume(k):
        ci = k * NW + wid
        b = k % 2
        pltpu.make_async_copy(src_emb.at[pl.ds(ci * CHUNK, CHUNK)], rows[b],
                              lsems[b]).wait()
        pltpu.make_async_copy(sectors.at[ci], idxs[b], isems[b]).wait()
        for j in range(NSUB):
            pltpu.sync_copy(rows[b].at[pl.ds(j * SUB, SUB)],
                            acc_sh.at[idxs[b].at[j]], add=True)

    start(0)
    for k in range(NCH_PER_W):
        if k + 1 < NCH_PER_W:
            start(k + 1)
        consume(k)

    plsc.subcore_barrier()

    @pl.when(s == 0)
    def _():
        pltpu.sync_copy(acc_sh, out_sums.at[c])


_seg_call_cache = []


def _seg_call(*args):
    if not _seg_call_cache:
        _seg_call_cache.append(pl.kernel(
            _seg_sum_body,
            out_type=jax.ShapeDtypeStruct((NC, NSEC_PAD, D), jnp.float32),
            mesh=plsc.VectorSubcoreMesh(core_axis_name="c",
                                        subcore_axis_name="s",
                                        num_cores=NC, num_subcores=NS),
            scratch_types=[
                pltpu.VMEM((CHUNK, D), jnp.float32),
                pltpu.VMEM((CHUNK, D), jnp.float32),
                pltpu.VMEM((NSUB, SUB), jnp.int32),
                pltpu.VMEM((NSUB, SUB), jnp.int32),
                pltpu.VMEM_SHARED((NSEC_PAD, D), jnp.float32),
                pltpu.SemaphoreType.DMA,
                pltpu.SemaphoreType.DMA,
                pltpu.SemaphoreType.DMA,
                pltpu.SemaphoreType.DMA,
            ],
        ))
    return _seg_call_cache[0](*args)


def _hist_body(sec_ref, out_ref):
    i = pl.program_id(0)

    @pl.when(i == 0)
    def _():
        out_ref[...] = jnp.zeros((1, NSEC_PAD), jnp.float32)

    sec = sec_ref[0, 0, :]
    onehot = (sec[:, None] == lax.broadcasted_iota(
        jnp.int32, (HBLK, NSEC_PAD), 1)).astype(jnp.bfloat16)
    ones = jnp.ones((1, HBLK), jnp.bfloat16)
    out_ref[...] += jnp.dot(ones, onehot,
                            preferred_element_type=jnp.float32)


def _hist_call(sec_r):
    return pl.pallas_call(
        _hist_body,
        grid=(HGRID,),
        in_specs=[pl.BlockSpec((1, 1, HBLK), lambda i: (i, 0, 0))],
        out_specs=pl.BlockSpec((1, NSEC_PAD), lambda i: (0, 0)),
        out_shape=jax.ShapeDtypeStruct((1, NSEC_PAD), jnp.float32),
    )(sec_r)


def _tcseg_body(rows_ref, sec_ref, out_ref):
    i = pl.program_id(0)

    @pl.when(i == 0)
    def _():
        out_ref[...] = jnp.zeros((NSEC_PAD, D), jnp.float32)

    sec = sec_ref[0, 0, :]
    oh_t = (lax.broadcasted_iota(jnp.int32, (NSEC_PAD, TSEG), 0)
            == sec[None, :]).astype(jnp.bfloat16)
    rows_bf = rows_ref[...].astype(jnp.bfloat16)
    out_ref[...] += jnp.dot(oh_t, rows_bf,
                            preferred_element_type=jnp.float32)


def _tcseg_call(src_emb, sec_r):
    return pl.pallas_call(
        _tcseg_body,
        grid=(TSEG_GRID,),
        in_specs=[
            pl.BlockSpec((TSEG, D), lambda i: (TSEG_OFF + i, 0)),
            pl.BlockSpec((1, 1, TSEG), lambda i: (TSEG_OFF + i, 0, 0)),
        ],
        out_specs=pl.BlockSpec((NSEC_PAD, D), lambda i: (0, 0)),
        out_shape=jax.ShapeDtypeStruct((NSEC_PAD, D), jnp.float32),
    )(src_emb, sec_r)


def _dist_body(sums_ref, tcsum_ref, cnts_ref, temb_ref, tidx_ref, out_ref,
               centers_scr):
    i = pl.program_id(0)

    @pl.when(i == 0)
    def _():
        ssum = sums_ref[0] + sums_ref[1] + tcsum_ref[...]
        cnt = cnts_ref[0]
        centers = ssum / jnp.maximum(cnt, 1.0)[:, None]
        centers_scr[...] = centers.astype(jnp.bfloat16)
        out_ref[...] = jnp.zeros((1, 1), jnp.float32)

    sec = tidx_ref[0, 0, :] // SLICE_RANGE
    onehot = (sec[:, None] == lax.broadcasted_iota(
        jnp.int32, (TBLK, NSEC_PAD), 1)).astype(jnp.bfloat16)
    cc = jnp.dot(onehot, centers_scr[...],
                 preferred_element_type=jnp.float32)
    diff = temb_ref[...] - cc + 1e-6
    sq = diff * diff
    ones = jnp.ones((NSEC_PAD, 8), jnp.float32)
    dist2 = jnp.dot(sq, ones, preferred_element_type=jnp.float32)
    dist = jnp.sqrt(dist2[:, 0])
    out_ref[...] += (jnp.sum(dist) * (1.0 / B)).reshape(1, 1)


def _dist_call(sums, tcsum, cnts, temb, tidx_r):
    return pl.pallas_call(
        _dist_body,
        grid=(GRID,),
        in_specs=[
            pl.BlockSpec((NC, NSEC_PAD, D), lambda i: (0, 0, 0)),
            pl.BlockSpec((NSEC_PAD, D), lambda i: (0, 0)),
            pl.BlockSpec((1, NSEC_PAD), lambda i: (0, 0)),
            pl.BlockSpec((TBLK, D), lambda i: (i, 0)),
            pl.BlockSpec((1, 1, TBLK), lambda i: (i, 0, 0)),
        ],
        out_specs=pl.BlockSpec((1, 1), lambda i: (0, 0)),
        out_shape=jax.ShapeDtypeStruct((1, 1), jnp.float32),
        scratch_shapes=[pltpu.VMEM((NSEC_PAD, D), jnp.bfloat16)],
    )(sums, tcsum, cnts, temb, tidx_r)


def kernel(target_embeddings, target_slice_idx, source_embeddings,
           source_slice_idx, source_sectors):
    del source_slice_idx
    sec32 = source_sectors.astype(jnp.int32)
    sectors_r = sec32.reshape(NCHUNKS, NSUB, SUB)
    zeros_acc = jnp.zeros((NSEC_PAD, D), jnp.float32)

    sums = _seg_call(source_embeddings, sectors_r, zeros_acc)

    tcsum = _tcseg_call(source_embeddings,
                        sec32.reshape(N_SRC // TSEG, 1, TSEG))

    sec_pad = jnp.concatenate(
        [sec32, jnp.full((N_SRC_PAD - N_SRC,), NSEC_PAD - 1, jnp.int32)])
    cnts = _hist_call(sec_pad.reshape(HGRID, 1, HBLK))

    tidx_r = target_slice_idx.astype(jnp.int32).reshape(GRID, 1, TBLK)
    out = _dist_call(sums, tcsum, cnts, target_embeddings, tidx_r)
    return out[0, 0]


# speedup vs baseline: 1.0998x; 1.0998x over previous
import jax
import jax.numpy as jnp
from jax import lax
from jax.experimental import pallas as pl
from jax.experimental.pallas import tpu as pltpu
from jax.experimental.pallas import tpu_sc as plsc

N_SRC = 100000
D = 128
NSEC = 100
NSEC_PAD = 128
B = 16384
SLICE_RANGE = 1000

NC = 2
NS = 16
NW = NC * NS

CHUNK = 400
SUB = 100
NSUB = CHUNK // SUB
NCHUNKS = N_SRC // CHUNK

N_SC = 64000
NCHUNKS_SC = N_SC // CHUNK
NCH_PER_W = NCHUNKS_SC // NW

TSEG = 4000
TSEG_GRID = (N_SRC - N_SC) // TSEG
TSEG_OFF = N_SC // TSEG

HBLK = 12544
HGRID = -(-N_SRC // HBLK)
N_SRC_PAD = HGRID * HBLK

TBLK = 2048
GRID = B // TBLK


def _seg_sum_body(src_emb, sectors, zeros_acc, out_sums,
                  rows_v0, rows_v1, idx_v0, idx_v1, acc_sh,
                  lsem0, lsem1, isem0, isem1):
    c = lax.axis_index("c")
    s = lax.axis_index("s")
    wid = c * NS + s

    @pl.when(s == 0)
    def _():
        pltpu.sync_copy(zeros_acc, acc_sh)

    plsc.subcore_barrier()

    rows = (rows_v0, rows_v1)
    idxs = (idx_v0, idx_v1)
    lsems = (lsem0, lsem1)
    isems = (isem0, isem1)

    def start(k):
        ci = k * NW + wid
        b = k % 2
        pltpu.async_copy(src_emb.at[pl.ds(ci * CHUNK, CHUNK)], rows[b],
                         lsems[b])
        pltpu.async_copy(sectors.at[ci], idxs[b], isems[b])

    def consume(k):
        ci = k * NW + wid
        b = k % 2
        pltpu.make_async_copy(src_emb.at[pl.ds(ci * CHUNK, CHUNK)], rows[b],
                              lsems[b]).wait()
        pltpu.make_async_copy(sectors.at[ci], idxs[b], isems[b]).wait()
        for j in range(NSUB):
            pltpu.sync_copy(rows[b].at[pl.ds(j * SUB, SUB)],
                            acc_sh.at[idxs[b].at[j]], add=True)

    start(0)
    for k in range(NCH_PER_W):
        if k + 1 < NCH_PER_W:
            start(k + 1)
        consume(k)

    plsc.subcore_barrier()

    @pl.when(s == 0)
    def _():
        pltpu.sync_copy(acc_sh, out_sums.at[c])


_seg_call_cache = []


def _seg_call(*args):
    if not _seg_call_cache:
        _seg_call_cache.append(pl.kernel(
            _seg_sum_body,
            out_type=jax.ShapeDtypeStruct((NC, NSEC_PAD, D), jnp.float32),
            mesh=plsc.VectorSubcoreMesh(core_axis_name="c",
                                        subcore_axis_name="s",
                                        num_cores=NC, num_subcores=NS),
            scratch_types=[
                pltpu.VMEM((CHUNK, D), jnp.float32),
                pltpu.VMEM((CHUNK, D), jnp.float32),
                pltpu.VMEM((NSUB, SUB), jnp.int32),
                pltpu.VMEM((NSUB, SUB), jnp.int32),
                pltpu.VMEM_SHARED((NSEC_PAD, D), jnp.float32),
                pltpu.SemaphoreType.DMA,
                pltpu.SemaphoreType.DMA,
                pltpu.SemaphoreType.DMA,
                pltpu.SemaphoreType.DMA,
            ],
        ))
    return _seg_call_cache[0](*args)


def _hist_body(sec_ref, out_ref):
    i = pl.program_id(0)

    @pl.when(i == 0)
    def _():
        out_ref[...] = jnp.zeros((1, NSEC_PAD), jnp.float32)

    sec = sec_ref[0, 0, :]
    onehot = (sec[:, None] == lax.broadcasted_iota(
        jnp.int32, (HBLK, NSEC_PAD), 1)).astype(jnp.float32)
    out_ref[...] += jnp.sum(onehot, axis=0).reshape(1, NSEC_PAD)


def _hist_call(sec_r):
    return pl.pallas_call(
        _hist_body,
        grid=(HGRID,),
        in_specs=[pl.BlockSpec((1, 1, HBLK), lambda i: (i, 0, 0))],
        out_specs=pl.BlockSpec((1, NSEC_PAD), lambda i: (0, 0)),
        out_shape=jax.ShapeDtypeStruct((1, NSEC_PAD), jnp.float32),
    )(sec_r)


def _tcseg_body(rows_ref, sec_ref, out_ref):
    i = pl.program_id(0)

    @pl.when(i == 0)
    def _():
        out_ref[...] = jnp.zeros((NSEC_PAD, D), jnp.float32)

    sec = sec_ref[0, 0, :]
    oh_t = (lax.broadcasted_iota(jnp.int32, (NSEC_PAD, TSEG), 0)
            == sec[None, :]).astype(jnp.bfloat16)
    rows_bf = rows_ref[...].astype(jnp.bfloat16)
    out_ref[...] += jnp.dot(oh_t, rows_bf,
                            preferred_element_type=jnp.float32)


def _tcseg_call(src_emb, sec_r):
    return pl.pallas_call(
        _tcseg_body,
        grid=(TSEG_GRID,),
        in_specs=[
            pl.BlockSpec((TSEG, D), lambda i: (TSEG_OFF + i, 0)),
            pl.BlockSpec((1, 1, TSEG), lambda i: (TSEG_OFF + i, 0, 0)),
        ],
        out_specs=pl.BlockSpec((NSEC_PAD, D), lambda i: (0, 0)),
        out_shape=jax.ShapeDtypeStruct((NSEC_PAD, D), jnp.float32),
    )(src_emb, sec_r)


def _dist_body(sums_ref, tcsum_ref, cnts_ref, temb_ref, tidx_ref, out_ref,
               centers_scr):
    i = pl.program_id(0)

    @pl.when(i == 0)
    def _():
        ssum = sums_ref[0] + sums_ref[1] + tcsum_ref[...]
        cnt = cnts_ref[0]
        centers = ssum / jnp.maximum(cnt, 1.0)[:, None]
        centers_scr[...] = centers.astype(jnp.bfloat16)
        out_ref[...] = jnp.zeros((1, 1), jnp.float32)

    sec = tidx_ref[0, 0, :] // SLICE_RANGE
    onehot = (sec[:, None] == lax.broadcasted_iota(
        jnp.int32, (TBLK, NSEC_PAD), 1)).astype(jnp.bfloat16)
    cc = jnp.dot(onehot, centers_scr[...],
                 preferred_element_type=jnp.float32)
    diff = temb_ref[...] - cc + 1e-6
    sq = (diff * diff).astype(jnp.bfloat16)
    ones = jnp.ones((NSEC_PAD, 8), jnp.bfloat16)
    dist2 = jnp.dot(sq, ones, preferred_element_type=jnp.float32)
    dist = jnp.sqrt(dist2[:, 0])
    out_ref[...] += (jnp.sum(dist) * (1.0 / B)).reshape(1, 1)


def _dist_call(sums, tcsum, cnts, temb, tidx_r):
    return pl.pallas_call(
        _dist_body,
        grid=(GRID,),
        in_specs=[
            pl.BlockSpec((NC, NSEC_PAD, D), lambda i: (0, 0, 0)),
            pl.BlockSpec((NSEC_PAD, D), lambda i: (0, 0)),
            pl.BlockSpec((1, NSEC_PAD), lambda i: (0, 0)),
            pl.BlockSpec((TBLK, D), lambda i: (i, 0)),
            pl.BlockSpec((1, 1, TBLK), lambda i: (i, 0, 0)),
        ],
        out_specs=pl.BlockSpec((1, 1), lambda i: (0, 0)),
        out_shape=jax.ShapeDtypeStruct((1, 1), jnp.float32),
        scratch_shapes=[pltpu.VMEM((NSEC_PAD, D), jnp.bfloat16)],
    )(sums, tcsum, cnts, temb, tidx_r)


def kernel(target_embeddings, target_slice_idx, source_embeddings,
           source_slice_idx, source_sectors):
    del source_slice_idx
    sec32 = source_sectors.astype(jnp.int32)
    sectors_r = sec32.reshape(NCHUNKS, NSUB, SUB)
    zeros_acc = jnp.zeros((NSEC_PAD, D), jnp.float32)

    sums = _seg_call(source_embeddings, sectors_r, zeros_acc)

    tcsum = _tcseg_call(source_embeddings,
                        sec32.reshape(N_SRC // TSEG, 1, TSEG))

    sec_pad = jnp.concatenate(
        [sec32, jnp.full((N_SRC_PAD - N_SRC,), NSEC_PAD - 1, jnp.int32)])
    cnts = _hist_call(sec_pad.reshape(HGRID, 1, HBLK))

    tidx_r = target_slice_idx.astype(jnp.int32).reshape(GRID, 1, TBLK)
    out = _dist_call(sums, tcsum, cnts, target_embeddings, tidx_r)
    return out[0, 0]


# speedup vs baseline: 1.2126x; 1.1026x over previous
import jax
import jax.numpy as jnp
from jax import lax
from jax.experimental import pallas as pl
from jax.experimental.pallas import tpu as pltpu
from jax.experimental.pallas import tpu_sc as plsc

N_SRC = 100000
D = 128
NSEC = 100
NSEC_PAD = 128
B = 16384
SLICE_RANGE = 1000

NC = 2
NS = 16
NW = NC * NS

CHUNK = 400
SUB = 100
NSUB = CHUNK // SUB
NCHUNKS = N_SRC // CHUNK

N_SC = 51200
NCHUNKS_SC = N_SC // CHUNK
NCH_PER_W = NCHUNKS_SC // NW

TSEG = 10000
TSEG_OFF = 5
TSEG_GRID = N_SRC // TSEG - TSEG_OFF

HBLK = 12544
HGRID = -(-N_SRC // HBLK)
N_SRC_PAD = HGRID * HBLK

TBLK = 8192
GRID = B // TBLK


def _seg_sum_body(src_emb, sectors, zeros_acc, out_sums,
                  rows_v0, rows_v1, idx_v0, idx_v1, acc_sh,
                  lsem0, lsem1, isem0, isem1):
    c = lax.axis_index("c")
    s = lax.axis_index("s")
    wid = c * NS + s

    @pl.when(s == 0)
    def _():
        pltpu.sync_copy(zeros_acc, acc_sh)

    plsc.subcore_barrier()

    rows = (rows_v0, rows_v1)
    idxs = (idx_v0, idx_v1)
    lsems = (lsem0, lsem1)
    isems = (isem0, isem1)

    def start(k):
        ci = k * NW + wid
        b = k % 2
        pltpu.async_copy(src_emb.at[pl.ds(ci * CHUNK, CHUNK)], rows[b],
                         lsems[b])
        pltpu.async_copy(sectors.at[ci], idxs[b], isems[b])

    def consume(k):
        ci = k * NW + wid
        b = k % 2
        pltpu.make_async_copy(src_emb.at[pl.ds(ci * CHUNK, CHUNK)], rows[b],
                              lsems[b]).wait()
        pltpu.make_async_copy(sectors.at[ci], idxs[b], isems[b]).wait()
        for j in range(NSUB):
            pltpu.sync_copy(rows[b].at[pl.ds(j * SUB, SUB)],
                            acc_sh.at[idxs[b].at[j]], add=True)

    start(0)
    for k in range(NCH_PER_W):
        if k + 1 < NCH_PER_W:
            start(k + 1)
        consume(k)

    plsc.subcore_barrier()

    @pl.when(s == 0)
    def _():
        pltpu.sync_copy(acc_sh, out_sums.at[c])


_seg_call_cache = []


def _seg_call(*args):
    if not _seg_call_cache:
        _seg_call_cache.append(pl.kernel(
            _seg_sum_body,
            out_type=jax.ShapeDtypeStruct((NC, NSEC_PAD, D), jnp.float32),
            mesh=plsc.VectorSubcoreMesh(core_axis_name="c",
                                        subcore_axis_name="s",
                                        num_cores=NC, num_subcores=NS),
            scratch_types=[
                pltpu.VMEM((CHUNK, D), jnp.float32),
                pltpu.VMEM((CHUNK, D), jnp.float32),
                pltpu.VMEM((NSUB, SUB), jnp.int32),
                pltpu.VMEM((NSUB, SUB), jnp.int32),
                pltpu.VMEM_SHARED((NSEC_PAD, D), jnp.float32),
                pltpu.SemaphoreType.DMA,
                pltpu.SemaphoreType.DMA,
                pltpu.SemaphoreType.DMA,
                pltpu.SemaphoreType.DMA,
            ],
        ))
    return _seg_call_cache[0](*args)


def _hist_body(sec_ref, out_ref):
    i = pl.program_id(0)

    @pl.when(i == 0)
    def _():
        out_ref[...] = jnp.zeros((NSEC_PAD, 8), jnp.float32)

    sec = sec_ref[0, 0, :]
    oh_t = (lax.broadcasted_iota(jnp.int32, (NSEC_PAD, HBLK), 0)
            == sec[None, :]).astype(jnp.bfloat16)
    ones = jnp.ones((HBLK, 8), jnp.bfloat16)
    out_ref[...] += jnp.dot(oh_t, ones,
                            preferred_element_type=jnp.float32)


def _hist_call(sec_r):
    return pl.pallas_call(
        _hist_body,
        grid=(HGRID,),
        in_specs=[pl.BlockSpec((1, 1, HBLK), lambda i: (i, 0, 0))],
        out_specs=pl.BlockSpec((NSEC_PAD, 8), lambda i: (0, 0)),
        out_shape=jax.ShapeDtypeStruct((NSEC_PAD, 8), jnp.float32),
    )(sec_r)


def _tcseg_body(rows_ref, sec_ref, out_ref):
    i = pl.program_id(0)

    @pl.when(i == 0)
    def _():
        out_ref[...] = jnp.zeros((NSEC_PAD, D), jnp.float32)

    base = (TSEG_OFF + i) * TSEG
    gidx = base + lax.broadcasted_iota(jnp.int32, (TSEG,), 0)
    sec = jnp.where(gidx >= N_SC, sec_ref[0, 0, :], -1)
    oh_t = (lax.broadcasted_iota(jnp.int32, (NSEC_PAD, TSEG), 0)
            == sec[None, :]).astype(jnp.bfloat16)
    rows_bf = rows_ref[...].astype(jnp.bfloat16)
    out_ref[...] += jnp.dot(oh_t, rows_bf,
                            preferred_element_type=jnp.float32)


def _tcseg_call(src_emb, sec_r):
    return pl.pallas_call(
        _tcseg_body,
        grid=(TSEG_GRID,),
        in_specs=[
            pl.BlockSpec((TSEG, D), lambda i: (TSEG_OFF + i, 0)),
            pl.BlockSpec((1, 1, TSEG), lambda i: (TSEG_OFF + i, 0, 0)),
        ],
        out_specs=pl.BlockSpec((NSEC_PAD, D), lambda i: (0, 0)),
        out_shape=jax.ShapeDtypeStruct((NSEC_PAD, D), jnp.float32),
    )(src_emb, sec_r)


def _dist_body(sums_ref, tcsum_ref, cnts_ref, temb_ref, tidx_ref, out_ref,
               centers_scr):
    i = pl.program_id(0)

    @pl.when(i == 0)
    def _():
        ssum = sums_ref[0] + sums_ref[1] + tcsum_ref[...]
        cnt = cnts_ref[:, 0]
        centers = ssum / jnp.maximum(cnt, 1.0)[:, None]
        centers_scr[...] = centers.astype(jnp.bfloat16)
        out_ref[...] = jnp.zeros((1, 1), jnp.float32)

    sec = tidx_ref[0, 0, :] // SLICE_RANGE
    onehot = (sec[:, None] == lax.broadcasted_iota(
        jnp.int32, (TBLK, NSEC_PAD), 1)).astype(jnp.bfloat16)
    cc = jnp.dot(onehot, centers_scr[...],
                 preferred_element_type=jnp.float32)
    diff = temb_ref[...] - cc + 1e-6
    sq = (diff * diff).astype(jnp.bfloat16)
    ones = jnp.ones((NSEC_PAD, 8), jnp.bfloat16)
    dist2 = jnp.dot(sq, ones, preferred_element_type=jnp.float32)
    dist = jnp.sqrt(dist2[:, 0])
    out_ref[...] += (jnp.sum(dist) * (1.0 / B)).reshape(1, 1)


def _dist_call(sums, tcsum, cnts, temb, tidx_r):
    return pl.pallas_call(
        _dist_body,
        grid=(GRID,),
        in_specs=[
            pl.BlockSpec((NC, NSEC_PAD, D), lambda i: (0, 0, 0)),
            pl.BlockSpec((NSEC_PAD, D), lambda i: (0, 0)),
            pl.BlockSpec((NSEC_PAD, 8), lambda i: (0, 0)),
            pl.BlockSpec((TBLK, D), lambda i: (i, 0)),
            pl.BlockSpec((1, 1, TBLK), lambda i: (i, 0, 0)),
        ],
        out_specs=pl.BlockSpec((1, 1), lambda i: (0, 0)),
        out_shape=jax.ShapeDtypeStruct((1, 1), jnp.float32),
        scratch_shapes=[pltpu.VMEM((NSEC_PAD, D), jnp.bfloat16)],
    )(sums, tcsum, cnts, temb, tidx_r)


def kernel(target_embeddings, target_slice_idx, source_embeddings,
           source_slice_idx, source_sectors):
    del source_slice_idx
    sec32 = source_sectors.astype(jnp.int32)
    sectors_r = sec32.reshape(NCHUNKS, NSUB, SUB)
    zeros_acc = jnp.zeros((NSEC_PAD, D), jnp.float32)

    sums = _seg_call(source_embeddings, sectors_r, zeros_acc)

    tcsum = _tcseg_call(source_embeddings,
                        sec32.reshape(N_SRC // TSEG, 1, TSEG))

    sec_pad = jnp.concatenate(
        [sec32, jnp.full((N_SRC_PAD - N_SRC,), NSEC_PAD - 1, jnp.int32)])
    cnts = _hist_call(sec_pad.reshape(HGRID, 1, HBLK))

    tidx_r = target_slice_idx.astype(jnp.int32).reshape(GRID, 1, TBLK)
    out = _dist_call(sums, tcsum, cnts, target_embeddings, tidx_r)
    return out[0, 0]


# speedup vs baseline: 1.3538x; 1.1165x over previous
import jax
import jax.numpy as jnp
from jax import lax
from jax.experimental import pallas as pl
from jax.experimental.pallas import tpu as pltpu
from jax.experimental.pallas import tpu_sc as plsc

N_SRC = 100000
D = 128
NSEC = 100
NSEC_PAD = 128
B = 16384
SLICE_RANGE = 1000

NC = 2
NS = 16
NW = NC * NS

CHUNK = 384
SUB = 96
NSUB = CHUNK // SUB

N_SC = 49152
NCHUNKS_SC = N_SC // CHUNK
NCH_PER_W = NCHUNKS_SC // NW

TSEG = 12544
TSEG_GRID = 8
TSEG_SUMS_START = 3

HBLK = 12544
HGRID = -(-N_SRC // HBLK)
N_SRC_PAD = HGRID * HBLK

TBLK = 8192
GRID = B // TBLK


def _seg_sum_body(src_emb, sectors, zeros_acc, out_sums,
                  rows_v0, rows_v1, idx_v0, idx_v1,
                  acc_sh, lsem0, lsem1, isem0, isem1):
    c = lax.axis_index("c")
    s = lax.axis_index("s")
    wid = c * NS + s

    @pl.when(s == 0)
    def _():
        pltpu.sync_copy(zeros_acc, acc_sh)

    plsc.subcore_barrier()

    rows = (rows_v0, rows_v1)
    idxs = (idx_v0, idx_v1)
    lsems = (lsem0, lsem1)
    isems = (isem0, isem1)

    def start(k):
        ci = k * NW + wid
        b = k % 2
        pltpu.async_copy(src_emb.at[pl.ds(ci * CHUNK, CHUNK)], rows[b],
                         lsems[b])
        for j in range(NSUB):
            pltpu.async_copy(sectors.at[pl.ds(ci * CHUNK + j * SUB, SUB)],
                             idxs[b].at[j], isems[b])

    def consume(k):
        ci = k * NW + wid
        b = k % 2
        pltpu.make_async_copy(src_emb.at[pl.ds(ci * CHUNK, CHUNK)], rows[b],
                              lsems[b]).wait()
        for j in range(NSUB):
            pltpu.make_async_copy(
                sectors.at[pl.ds(ci * CHUNK + j * SUB, SUB)],
                idxs[b].at[j], isems[b]).wait()
        for j in range(NSUB):
            pltpu.sync_copy(rows[b].at[pl.ds(j * SUB, SUB)],
                            acc_sh.at[idxs[b].at[j]], add=True)

    start(0)
    for k in range(NCH_PER_W):
        if k + 1 < NCH_PER_W:
            start(k + 1)
        consume(k)

    plsc.subcore_barrier()

    @pl.when(s == 0)
    def _():
        pltpu.sync_copy(acc_sh, out_sums.at[c])


_seg_call_cache = []


def _seg_call(*args):
    if not _seg_call_cache:
        _seg_call_cache.append(pl.kernel(
            _seg_sum_body,
            out_type=jax.ShapeDtypeStruct((NC, NSEC_PAD, D), jnp.float32),
            mesh=plsc.VectorSubcoreMesh(core_axis_name="c",
                                        subcore_axis_name="s",
                                        num_cores=NC, num_subcores=NS),
            scratch_types=[
                pltpu.VMEM((CHUNK, D), jnp.float32),
                pltpu.VMEM((CHUNK, D), jnp.float32),
                pltpu.VMEM((NSUB, SUB), jnp.int32),
                pltpu.VMEM((NSUB, SUB), jnp.int32),
                pltpu.VMEM_SHARED((NSEC_PAD, D), jnp.float32),
                pltpu.SemaphoreType.DMA,
                pltpu.SemaphoreType.DMA,
                pltpu.SemaphoreType.DMA,
                pltpu.SemaphoreType.DMA,
            ],
        ))
    return _seg_call_cache[0](*args)


def _tcseg_body(rows_ref, sec_ref, sums_ref, cnts_ref):
    i = pl.program_id(0)

    @pl.when(i == 0)
    def _():
        sums_ref[...] = jnp.zeros((NSEC_PAD, D), jnp.float32)
        cnts_ref[...] = jnp.zeros((NSEC_PAD, 8), jnp.float32)

    sec = sec_ref[0, 0, :]
    oh_cnt = (lax.broadcasted_iota(jnp.int32, (NSEC_PAD, TSEG), 0)
              == sec[None, :]).astype(jnp.bfloat16)
    ones = jnp.ones((TSEG, 8), jnp.bfloat16)
    cnts_ref[...] += jnp.dot(oh_cnt, ones,
                             preferred_element_type=jnp.float32)

    @pl.when(i >= TSEG_SUMS_START)
    def _():
        gidx = i * TSEG + lax.broadcasted_iota(jnp.int32, (TSEG,), 0)
        sec2 = jnp.where(gidx >= N_SC, sec, -1)
        oh_t = (lax.broadcasted_iota(jnp.int32, (NSEC_PAD, TSEG), 0)
                == sec2[None, :]).astype(jnp.bfloat16)
        gidx2 = i * TSEG + lax.broadcasted_iota(jnp.int32, (TSEG, 1), 0)
        rows_bf = jnp.where(gidx2 < N_SRC, rows_ref[...],
                            0.0).astype(jnp.bfloat16)
        sums_ref[...] += jnp.dot(oh_t, rows_bf,
                                 preferred_element_type=jnp.float32)


def _tcseg_call(src_emb, sec_r):
    return pl.pallas_call(
        _tcseg_body,
        grid=(TSEG_GRID,),
        in_specs=[
            pl.BlockSpec((TSEG, D),
                         lambda i: (jnp.maximum(i, TSEG_SUMS_START), 0)),
            pl.BlockSpec((1, 1, TSEG), lambda i: (i, 0, 0)),
        ],
        out_specs=[
            pl.BlockSpec((NSEC_PAD, D), lambda i: (0, 0)),
            pl.BlockSpec((NSEC_PAD, 8), lambda i: (0, 0)),
        ],
        out_shape=[
            jax.ShapeDtypeStruct((NSEC_PAD, D), jnp.float32),
            jax.ShapeDtypeStruct((NSEC_PAD, 8), jnp.float32),
        ],
    )(src_emb, sec_r)


def _dist_body(sums_ref, tcsum_ref, cnts_ref, temb_ref, tidx_ref, out_ref,
               centers_scr):
    i = pl.program_id(0)

    @pl.when(i == 0)
    def _():
        ssum = sums_ref[0] + sums_ref[1] + tcsum_ref[...]
        cnt = cnts_ref[:, 0]
        centers = ssum / jnp.maximum(cnt, 1.0)[:, None]
        centers_scr[...] = centers.astype(jnp.bfloat16)
        out_ref[...] = jnp.zeros((1, 1), jnp.float32)

    sec = tidx_ref[0, 0, :] // SLICE_RANGE
    onehot = (sec[:, None] == lax.broadcasted_iota(
        jnp.int32, (TBLK, NSEC_PAD), 1)).astype(jnp.bfloat16)
    cc = jnp.dot(onehot, centers_scr[...],
                 preferred_element_type=jnp.float32)
    diff = temb_ref[...] - cc + 1e-6
    sq = (diff * diff).astype(jnp.bfloat16)
    ones = jnp.ones((NSEC_PAD, 8), jnp.bfloat16)
    dist2 = jnp.dot(sq, ones, preferred_element_type=jnp.float32)
    dist = jnp.sqrt(dist2[:, 0])
    out_ref[...] += (jnp.sum(dist) * (1.0 / B)).reshape(1, 1)


def _dist_call(sums, tcsum, cnts, temb, tidx_r):
    return pl.pallas_call(
        _dist_body,
        grid=(GRID,),
        in_specs=[
            pl.BlockSpec((NC, NSEC_PAD, D), lambda i: (0, 0, 0)),
            pl.BlockSpec((NSEC_PAD, D), lambda i: (0, 0)),
            pl.BlockSpec((NSEC_PAD, 8), lambda i: (0, 0)),
            pl.BlockSpec((TBLK, D), lambda i: (i, 0)),
            pl.BlockSpec((1, 1, TBLK), lambda i: (i, 0, 0)),
        ],
        out_specs=pl.BlockSpec((1, 1), lambda i: (0, 0)),
        out_shape=jax.ShapeDtypeStruct((1, 1), jnp.float32),
        scratch_shapes=[pltpu.VMEM((NSEC_PAD, D), jnp.bfloat16)],
    )(sums, tcsum, cnts, temb, tidx_r)


def kernel(target_embeddings, target_slice_idx, source_embeddings,
           source_slice_idx, source_sectors):
    del source_slice_idx
    sec32 = source_sectors.astype(jnp.int32)
    zeros_acc = jnp.zeros((NSEC_PAD, D), jnp.float32)

    sums = _seg_call(source_embeddings, sec32, zeros_acc)

    sec_pad = jnp.concatenate(
        [sec32, jnp.full((N_SRC_PAD - N_SRC,), NSEC_PAD - 1, jnp.int32)])
    tcsum, cnts = _tcseg_call(source_embeddings,
                              sec_pad.reshape(TSEG_GRID, 1, TSEG))

    tidx_r = target_slice_idx.astype(jnp.int32).reshape(GRID, 1, TBLK)
    out = _dist_call(sums, tcsum, cnts, target_embeddings, tidx_r)
    return out[0, 0]


# speedup vs baseline: 1.3587x; 1.0036x over previous
import jax
import jax.numpy as jnp
from jax import lax
from jax.experimental import pallas as pl
from jax.experimental.pallas import tpu as pltpu
from jax.experimental.pallas import tpu_sc as plsc

N_SRC = 100000
D = 128
NSEC = 100
NSEC_PAD = 128
B = 16384
SLICE_RANGE = 1000

NC = 2
NS = 16
NW = NC * NS

CHUNK = 384
SUB = 96
NSUB = CHUNK // SUB

N_SC = 49152
NCHUNKS_SC = N_SC // CHUNK
NCH_PER_W = NCHUNKS_SC // NW

TSEG = 12544
TSEG_GRID = 8
TSEG_SUMS_START = 3

HBLK = 12544
HGRID = -(-N_SRC // HBLK)
N_SRC_PAD = HGRID * HBLK

TBLK = 8192
GRID = B // TBLK


def _seg_sum_body(src_emb, sectors, zeros_acc, out_sums,
                  rows_v0, rows_v1, idx_v0, idx_v1,
                  acc_sh, lsem0, lsem1, isem0, isem1):
    c = lax.axis_index("c")
    s = lax.axis_index("s")
    wid = c * NS + s

    @pl.when(s == 0)
    def _():
        pltpu.sync_copy(zeros_acc, acc_sh)

    plsc.subcore_barrier()

    rows = (rows_v0, rows_v1)
    idxs = (idx_v0, idx_v1)
    lsems = (lsem0, lsem1)
    isems = (isem0, isem1)

    def start(k):
        ci = k * NW + wid
        b = k % 2
        pltpu.async_copy(src_emb.at[pl.ds(ci * CHUNK, CHUNK)], rows[b],
                         lsems[b])
        for j in range(NSUB):
            pltpu.async_copy(sectors.at[pl.ds(ci * CHUNK + j * SUB, SUB)],
                             idxs[b].at[j], isems[b])

    def consume(k):
        ci = k * NW + wid
        b = k % 2
        pltpu.make_async_copy(src_emb.at[pl.ds(ci * CHUNK, CHUNK)], rows[b],
                              lsems[b]).wait()
        for j in range(NSUB):
            pltpu.make_async_copy(
                sectors.at[pl.ds(ci * CHUNK + j * SUB, SUB)],
                idxs[b].at[j], isems[b]).wait()
        for j in range(NSUB):
            pltpu.sync_copy(rows[b].at[pl.ds(j * SUB, SUB)],
                            acc_sh.at[idxs[b].at[j]], add=True)

    start(0)
    for k in range(NCH_PER_W):
        if k + 1 < NCH_PER_W:
            start(k + 1)
        consume(k)

    plsc.subcore_barrier()

    @pl.when(s == 0)
    def _():
        pltpu.sync_copy(acc_sh, out_sums.at[c])


_seg_call_cache = []


def _seg_call(*args):
    if not _seg_call_cache:
        _seg_call_cache.append(pl.kernel(
            _seg_sum_body,
            out_type=jax.ShapeDtypeStruct((NC, NSEC_PAD, D), jnp.float32),
            mesh=plsc.VectorSubcoreMesh(core_axis_name="c",
                                        subcore_axis_name="s",
                                        num_cores=NC, num_subcores=NS),
            scratch_types=[
                pltpu.VMEM((CHUNK, D), jnp.float32),
                pltpu.VMEM((CHUNK, D), jnp.float32),
                pltpu.VMEM((NSUB, SUB), jnp.int32),
                pltpu.VMEM((NSUB, SUB), jnp.int32),
                pltpu.VMEM_SHARED((NSEC_PAD, D), jnp.float32),
                pltpu.SemaphoreType.DMA,
                pltpu.SemaphoreType.DMA,
                pltpu.SemaphoreType.DMA,
                pltpu.SemaphoreType.DMA,
            ],
        ))
    return _seg_call_cache[0](*args)


def _tcseg_body(rows_ref, sec_ref, sums_ref, cnts_ref):
    i = pl.program_id(0)

    @pl.when(i == 0)
    def _():
        sums_ref[...] = jnp.zeros((NSEC_PAD, D), jnp.float32)
        cnts_ref[...] = jnp.zeros((NSEC_PAD, 8), jnp.float32)

    sec = sec_ref[0, 0, :]
    oh_cnt = (lax.broadcasted_iota(jnp.int32, (NSEC_PAD, TSEG), 0)
              == sec[None, :]).astype(jnp.bfloat16)
    ones = jnp.ones((TSEG, 8), jnp.bfloat16)
    cnts_ref[...] += jnp.dot(oh_cnt, ones,
                             preferred_element_type=jnp.float32)

    @pl.when(i >= TSEG_SUMS_START)
    def _():
        gidx = i * TSEG + lax.broadcasted_iota(jnp.int32, (TSEG,), 0)
        sec2 = jnp.where(gidx >= N_SC, sec, -1)
        oh_t = (lax.broadcasted_iota(jnp.int32, (NSEC_PAD, TSEG), 0)
                == sec2[None, :]).astype(jnp.bfloat16)
        gidx2 = i * TSEG + lax.broadcasted_iota(jnp.int32, (TSEG, 1), 0)
        rows_bf = jnp.where(gidx2 < N_SRC, rows_ref[...],
                            0.0).astype(jnp.bfloat16)
        sums_ref[...] += jnp.dot(oh_t, rows_bf,
                                 preferred_element_type=jnp.float32)


def _tcseg_call(src_emb, sec_r):
    return pl.pallas_call(
        _tcseg_body,
        grid=(TSEG_GRID,),
        in_specs=[
            pl.BlockSpec((TSEG, D),
                         lambda i: (jnp.maximum(i, TSEG_SUMS_START), 0)),
            pl.BlockSpec((1, 1, TSEG), lambda i: (i, 0, 0)),
        ],
        out_specs=[
            pl.BlockSpec((NSEC_PAD, D), lambda i: (0, 0)),
            pl.BlockSpec((NSEC_PAD, 8), lambda i: (0, 0)),
        ],
        out_shape=[
            jax.ShapeDtypeStruct((NSEC_PAD, D), jnp.float32),
            jax.ShapeDtypeStruct((NSEC_PAD, 8), jnp.float32),
        ],
    )(src_emb, sec_r)


def _dist_body(sums_ref, tcsum_ref, cnts_ref, temb_ref, tidx_ref, out_ref,
               centers_scr):
    i = pl.program_id(0)

    @pl.when(i == 0)
    def _():
        ssum = sums_ref[0] + sums_ref[1] + tcsum_ref[...]
        cnt = cnts_ref[:, 0]
        centers = ssum / jnp.maximum(cnt, 1.0)[:, None]
        centers_scr[...] = centers.astype(jnp.bfloat16)
        out_ref[...] = jnp.zeros((1, 1), jnp.float32)

    sec = tidx_ref[0, 0, :] // SLICE_RANGE
    onehot = (sec[:, None] == lax.broadcasted_iota(
        jnp.int32, (TBLK, NSEC_PAD), 1)).astype(jnp.bfloat16)
    cc = jnp.dot(onehot, centers_scr[...],
                 preferred_element_type=jnp.float32).astype(jnp.bfloat16)
    diff = temb_ref[...].astype(jnp.bfloat16) - cc
    sq = diff * diff
    ones = jnp.ones((NSEC_PAD, 8), jnp.bfloat16)
    dist2 = jnp.dot(sq, ones, preferred_element_type=jnp.float32)
    dist = jnp.sqrt(dist2[:, 0])
    out_ref[...] += (jnp.sum(dist) * (1.0 / B)).reshape(1, 1)


def _dist_call(sums, tcsum, cnts, temb, tidx_r):
    return pl.pallas_call(
        _dist_body,
        grid=(GRID,),
        in_specs=[
            pl.BlockSpec((NC, NSEC_PAD, D), lambda i: (0, 0, 0)),
            pl.BlockSpec((NSEC_PAD, D), lambda i: (0, 0)),
            pl.BlockSpec((NSEC_PAD, 8), lambda i: (0, 0)),
            pl.BlockSpec((TBLK, D), lambda i: (i, 0)),
            pl.BlockSpec((1, 1, TBLK), lambda i: (i, 0, 0)),
        ],
        out_specs=pl.BlockSpec((1, 1), lambda i: (0, 0)),
        out_shape=jax.ShapeDtypeStruct((1, 1), jnp.float32),
        scratch_shapes=[pltpu.VMEM((NSEC_PAD, D), jnp.bfloat16)],
    )(sums, tcsum, cnts, temb, tidx_r)


def kernel(target_embeddings, target_slice_idx, source_embeddings,
           source_slice_idx, source_sectors):
    del source_slice_idx
    sec32 = source_sectors.astype(jnp.int32)
    zeros_acc = jnp.zeros((NSEC_PAD, D), jnp.float32)

    sums = _seg_call(source_embeddings, sec32, zeros_acc)

    sec_pad = jnp.concatenate(
        [sec32, jnp.full((N_SRC_PAD - N_SRC,), NSEC_PAD - 1, jnp.int32)])
    tcsum, cnts = _tcseg_call(source_embeddings,
                              sec_pad.reshape(TSEG_GRID, 1, TSEG))

    tidx_r = target_slice_idx.astype(jnp.int32).reshape(GRID, 1, TBLK)
    out = _dist_call(sums, tcsum, cnts, target_embeddings, tidx_r)
    return out[0, 0]


# speedup vs baseline: 1.3736x; 1.0110x over previous
import jax
import jax.numpy as jnp
from jax import lax
from jax.experimental import pallas as pl
from jax.experimental.pallas import tpu as pltpu
from jax.experimental.pallas import tpu_sc as plsc

N_SRC = 100000
D = 128
NSEC = 100
NSEC_PAD = 128
B = 16384
SLICE_RANGE = 1000

NC = 2
NS = 16
NW = NC * NS

CHUNK = 384
SUB = 96
NSUB = CHUNK // SUB

N_SC = 49152
NCHUNKS_SC = N_SC // CHUNK
NCH_PER_W = NCHUNKS_SC // NW

TSEG = 12288
TSEG_GRID = 9
TSEG_SUMS_START = 4

TBLK = 4096
GRID = B // TBLK


def _seg_sum_body(src_emb, sectors, zeros_acc, out_sums,
                  rows_v0, rows_v1, idx_v0, idx_v1,
                  acc_sh, lsem0, lsem1, isem0, isem1):
    c = lax.axis_index("c")
    s = lax.axis_index("s")
    wid = c * NS + s

    @pl.when(s == 0)
    def _():
        pltpu.sync_copy(zeros_acc, acc_sh)

    plsc.subcore_barrier()

    rows = (rows_v0, rows_v1)
    idxs = (idx_v0, idx_v1)
    lsems = (lsem0, lsem1)
    isems = (isem0, isem1)

    def start(k):
        ci = k * NW + wid
        b = k % 2
        pltpu.async_copy(src_emb.at[pl.ds(ci * CHUNK, CHUNK)], rows[b],
                         lsems[b])
        for j in range(NSUB):
            pltpu.async_copy(sectors.at[pl.ds(ci * CHUNK + j * SUB, SUB)],
                             idxs[b].at[j], isems[b])

    def consume(k):
        ci = k * NW + wid
        b = k % 2
        pltpu.make_async_copy(src_emb.at[pl.ds(ci * CHUNK, CHUNK)], rows[b],
                              lsems[b]).wait()
        for j in range(NSUB):
            pltpu.make_async_copy(
                sectors.at[pl.ds(ci * CHUNK + j * SUB, SUB)],
                idxs[b].at[j], isems[b]).wait()
        for j in range(NSUB):
            pltpu.sync_copy(rows[b].at[pl.ds(j * SUB, SUB)],
                            acc_sh.at[idxs[b].at[j]], add=True)

    start(0)
    for k in range(NCH_PER_W):
        if k + 1 < NCH_PER_W:
            start(k + 1)
        consume(k)

    plsc.subcore_barrier()

    @pl.when(s == 0)
    def _():
        pltpu.sync_copy(acc_sh, out_sums.at[c])


_seg_call_cache = []


def _seg_call(*args):
    if not _seg_call_cache:
        _seg_call_cache.append(pl.kernel(
            _seg_sum_body,
            out_type=jax.ShapeDtypeStruct((NC, NSEC_PAD, D), jnp.float32),
            mesh=plsc.VectorSubcoreMesh(core_axis_name="c",
                                        subcore_axis_name="s",
                                        num_cores=NC, num_subcores=NS),
            scratch_types=[
                pltpu.VMEM((CHUNK, D), jnp.float32),
                pltpu.VMEM((CHUNK, D), jnp.float32),
                pltpu.VMEM((NSUB, SUB), jnp.int32),
                pltpu.VMEM((NSUB, SUB), jnp.int32),
                pltpu.VMEM_SHARED((NSEC_PAD, D), jnp.float32),
                pltpu.SemaphoreType.DMA,
                pltpu.SemaphoreType.DMA,
                pltpu.SemaphoreType.DMA,
                pltpu.SemaphoreType.DMA,
            ],
        ))
    return _seg_call_cache[0](*args)


def _tcseg_body(rows_ref, sec_ref, sums_ref, cnts_ref):
    i = pl.program_id(0)

    @pl.when(i == 0)
    def _():
        sums_ref[...] = jnp.zeros((NSEC_PAD, D), jnp.float32)
        cnts_ref[...] = jnp.zeros((NSEC_PAD, 8), jnp.float32)

    sec = sec_ref[0, 0, :]
    oh_cnt = (lax.broadcasted_iota(jnp.int32, (NSEC_PAD, TSEG), 0)
              == sec[None, :]).astype(jnp.bfloat16)
    ones = jnp.ones((TSEG, 8), jnp.bfloat16)
    cnts_ref[...] += jnp.dot(oh_cnt, ones,
                             preferred_element_type=jnp.float32)

    @pl.when(i >= TSEG_SUMS_START)
    def _():
        gidx = i * TSEG + lax.broadcasted_iota(jnp.int32, (TSEG,), 0)
        sec2 = jnp.where(gidx >= N_SC, sec, -1)
        oh_t = (lax.broadcasted_iota(jnp.int32, (NSEC_PAD, TSEG), 0)
                == sec2[None, :]).astype(jnp.bfloat16)
        gidx2 = i * TSEG + lax.broadcasted_iota(jnp.int32, (TSEG, 1), 0)
        rows_bf = jnp.where(gidx2 < N_SRC, rows_ref[...],
                            0.0).astype(jnp.bfloat16)
        sums_ref[...] += jnp.dot(oh_t, rows_bf,
                                 preferred_element_type=jnp.float32)


def _tcseg_call(src_emb, sec_r):
    return pl.pallas_call(
        _tcseg_body,
        grid=(TSEG_GRID,),
        in_specs=[
            pl.BlockSpec((TSEG, D),
                         lambda i: (jnp.maximum(i, TSEG_SUMS_START), 0)),
            pl.BlockSpec((1, 1, TSEG), lambda i: (i, 0, 0)),
        ],
        out_specs=[
            pl.BlockSpec((NSEC_PAD, D), lambda i: (0, 0)),
            pl.BlockSpec((NSEC_PAD, 8), lambda i: (0, 0)),
        ],
        out_shape=[
            jax.ShapeDtypeStruct((NSEC_PAD, D), jnp.float32),
            jax.ShapeDtypeStruct((NSEC_PAD, 8), jnp.float32),
        ],
    )(src_emb, sec_r)


def _dist_body(sums_ref, tcsum_ref, cnts_ref, temb_ref, tidx_ref, out_ref,
               centers_scr):
    i = pl.program_id(0)

    @pl.when(i == 0)
    def _():
        ssum = sums_ref[0] + sums_ref[1] + tcsum_ref[...]
        cnt = cnts_ref[:, 0]
        centers = ssum / jnp.maximum(cnt, 1.0)[:, None]
        centers_scr[...] = centers.astype(jnp.bfloat16)
        out_ref[...] = jnp.zeros((1, 1), jnp.float32)

    sec = tidx_ref[0, 0, :] // SLICE_RANGE
    onehot = (sec[:, None] == lax.broadcasted_iota(
        jnp.int32, (TBLK, NSEC_PAD), 1)).astype(jnp.bfloat16)
    cc = jnp.dot(onehot, centers_scr[...],
                 preferred_element_type=jnp.float32).astype(jnp.bfloat16)
    diff = temb_ref[...].astype(jnp.bfloat16) - cc
    sq = diff * diff
    ones = jnp.ones((NSEC_PAD, 8), jnp.bfloat16)
    dist2 = jnp.dot(sq, ones, preferred_element_type=jnp.float32)
    dist = jnp.sqrt(dist2[:, 0])
    out_ref[...] += (jnp.sum(dist) * (1.0 / B)).reshape(1, 1)


def _dist_call(sums, tcsum, cnts, temb, tidx_r):
    return pl.pallas_call(
        _dist_body,
        grid=(GRID,),
        in_specs=[
            pl.BlockSpec((NC, NSEC_PAD, D), lambda i: (0, 0, 0)),
            pl.BlockSpec((NSEC_PAD, D), lambda i: (0, 0)),
            pl.BlockSpec((NSEC_PAD, 8), lambda i: (0, 0)),
            pl.BlockSpec((TBLK, D), lambda i: (i, 0)),
            pl.BlockSpec((1, 1, TBLK), lambda i: (i, 0, 0)),
        ],
        out_specs=pl.BlockSpec((1, 1), lambda i: (0, 0)),
        out_shape=jax.ShapeDtypeStruct((1, 1), jnp.float32),
        scratch_shapes=[pltpu.VMEM((NSEC_PAD, D), jnp.bfloat16)],
    )(sums, tcsum, cnts, temb, tidx_r)


def kernel(target_embeddings, target_slice_idx, source_embeddings,
           source_slice_idx, source_sectors):
    del source_slice_idx
    sec32 = source_sectors.astype(jnp.int32)
    zeros_acc = jnp.zeros((NSEC_PAD, D), jnp.float32)

    sums = _seg_call(source_embeddings, sec32, zeros_acc)

    sec_pad = jnp.concatenate(
        [sec32, jnp.full((TSEG_GRID * TSEG - N_SRC,), NSEC_PAD - 1,
                         jnp.int32)])
    tcsum, cnts = _tcseg_call(source_embeddings,
                              sec_pad.reshape(TSEG_GRID, 1, TSEG))

    tidx_r = target_slice_idx.astype(jnp.int32).reshape(GRID, 1, TBLK)
    out = _dist_call(sums, tcsum, cnts, target_embeddings, tidx_r)
    return out[0, 0]


# speedup vs baseline: 1.3737x; 1.0000x over previous
import jax
import jax.numpy as jnp
from jax import lax
from jax.experimental import pallas as pl
from jax.experimental.pallas import tpu as pltpu
from jax.experimental.pallas import tpu_sc as plsc

N_SRC = 100000
D = 128
NSEC = 100
NSEC_PAD = 128
B = 16384
SLICE_RANGE = 1000

NC = 2
NS = 16
NW = NC * NS

CHUNK = 384
SUB = 96
NSUB = CHUNK // SUB

N_SC = 49152
NCHUNKS_SC = N_SC // CHUNK
NCH_PER_W = NCHUNKS_SC // NW

TSEG = 12288
TSEG_GRID = 9
TSEG_SUMS_START = 4

TBLK = 4096
GRID = B // TBLK


def _seg_sum_body(src_emb, sectors, zeros_acc, out_sums,
                  rows_v0, rows_v1, idx_v0, idx_v1,
                  acc_sh, lsem0, lsem1, isem0, isem1, ssem0, ssem1):
    c = lax.axis_index("c")
    s = lax.axis_index("s")
    wid = c * NS + s

    @pl.when(s == 0)
    def _():
        pltpu.sync_copy(zeros_acc, acc_sh)

    plsc.subcore_barrier()

    rows = (rows_v0, rows_v1)
    idxs = (idx_v0, idx_v1)
    lsems = (lsem0, lsem1)
    isems = (isem0, isem1)
    ssems = (ssem0, ssem1)

    def start(k):
        ci = k * NW + wid
        b = k % 2
        pltpu.async_copy(src_emb.at[pl.ds(ci * CHUNK, CHUNK)], rows[b],
                         lsems[b])
        for j in range(NSUB):
            pltpu.async_copy(sectors.at[pl.ds(ci * CHUNK + j * SUB, SUB)],
                             idxs[b].at[j], isems[b])

    def consume(k):
        ci = k * NW + wid
        b = k % 2
        pltpu.make_async_copy(src_emb.at[pl.ds(ci * CHUNK, CHUNK)], rows[b],
                              lsems[b]).wait()
        for j in range(NSUB):
            pltpu.make_async_copy(
                sectors.at[pl.ds(ci * CHUNK + j * SUB, SUB)],
                idxs[b].at[j], isems[b]).wait()
        for j in range(NSUB):
            pltpu.async_copy(rows[b].at[pl.ds(j * SUB, SUB)],
                             acc_sh.at[idxs[b].at[j]], ssems[b], add=True)

    def drain_scatters(k):
        b = k % 2
        for j in range(NSUB):
            pltpu.make_async_copy(rows[b].at[pl.ds(j * SUB, SUB)],
                                  acc_sh.at[idxs[b].at[j]], ssems[b]).wait()

    start(0)
    for k in range(NCH_PER_W):
        if k + 1 < NCH_PER_W:
            if k - 1 >= 0:
                drain_scatters(k - 1)
            start(k + 1)
        consume(k)
    if NCH_PER_W >= 2:
        drain_scatters(NCH_PER_W - 2)
    drain_scatters(NCH_PER_W - 1)

    plsc.subcore_barrier()

    @pl.when(s == 0)
    def _():
        pltpu.sync_copy(acc_sh, out_sums.at[c])


_seg_call_cache = []


def _seg_call(*args):
    if not _seg_call_cache:
        _seg_call_cache.append(pl.kernel(
            _seg_sum_body,
            out_type=jax.ShapeDtypeStruct((NC, NSEC_PAD, D), jnp.float32),
            mesh=plsc.VectorSubcoreMesh(core_axis_name="c",
                                        subcore_axis_name="s",
                                        num_cores=NC, num_subcores=NS),
            scratch_types=[
                pltpu.VMEM((CHUNK, D), jnp.float32),
                pltpu.VMEM((CHUNK, D), jnp.float32),
                pltpu.VMEM((NSUB, SUB), jnp.int32),
                pltpu.VMEM((NSUB, SUB), jnp.int32),
                pltpu.VMEM_SHARED((NSEC_PAD, D), jnp.float32),
                pltpu.SemaphoreType.DMA,
                pltpu.SemaphoreType.DMA,
                pltpu.SemaphoreType.DMA,
                pltpu.SemaphoreType.DMA,
                pltpu.SemaphoreType.DMA,
                pltpu.SemaphoreType.DMA,
            ],
        ))
    return _seg_call_cache[0](*args)


def _tcseg_body(rows_ref, sec_ref, sums_ref, cnts_ref):
    i = pl.program_id(0)

    @pl.when(i == 0)
    def _():
        sums_ref[...] = jnp.zeros((NSEC_PAD, D), jnp.float32)
        cnts_ref[...] = jnp.zeros((NSEC_PAD, 8), jnp.float32)

    sec = sec_ref[0, 0, :]
    oh_cnt = (lax.broadcasted_iota(jnp.int32, (NSEC_PAD, TSEG), 0)
              == sec[None, :]).astype(jnp.bfloat16)
    ones = jnp.ones((TSEG, 8), jnp.bfloat16)
    cnts_ref[...] += jnp.dot(oh_cnt, ones,
                             preferred_element_type=jnp.float32)

    @pl.when(i >= TSEG_SUMS_START)
    def _():
        gidx = i * TSEG + lax.broadcasted_iota(jnp.int32, (TSEG,), 0)
        sec2 = jnp.where(gidx >= N_SC, sec, -1)
        oh_t = (lax.broadcasted_iota(jnp.int32, (NSEC_PAD, TSEG), 0)
                == sec2[None, :]).astype(jnp.bfloat16)
        gidx2 = i * TSEG + lax.broadcasted_iota(jnp.int32, (TSEG, 1), 0)
        rows_bf = jnp.where(gidx2 < N_SRC, rows_ref[...],
                            0.0).astype(jnp.bfloat16)
        sums_ref[...] += jnp.dot(oh_t, rows_bf,
                                 preferred_element_type=jnp.float32)


def _tcseg_call(src_emb, sec_r):
    return pl.pallas_call(
        _tcseg_body,
        grid=(TSEG_GRID,),
        in_specs=[
            pl.BlockSpec((TSEG, D),
                         lambda i: (jnp.maximum(i, TSEG_SUMS_START), 0)),
            pl.BlockSpec((1, 1, TSEG), lambda i: (i, 0, 0)),
        ],
        out_specs=[
            pl.BlockSpec((NSEC_PAD, D), lambda i: (0, 0)),
            pl.BlockSpec((NSEC_PAD, 8), lambda i: (0, 0)),
        ],
        out_shape=[
            jax.ShapeDtypeStruct((NSEC_PAD, D), jnp.float32),
            jax.ShapeDtypeStruct((NSEC_PAD, 8), jnp.float32),
        ],
    )(src_emb, sec_r)


def _dist_body(sums_ref, tcsum_ref, cnts_ref, temb_ref, tidx_ref, out_ref,
               centers_scr):
    i = pl.program_id(0)

    @pl.when(i == 0)
    def _():
        ssum = sums_ref[0] + sums_ref[1] + tcsum_ref[...]
        cnt = cnts_ref[:, 0]
        centers = ssum / jnp.maximum(cnt, 1.0)[:, None]
        centers_scr[...] = centers.astype(jnp.bfloat16)
        out_ref[...] = jnp.zeros((1, 1), jnp.float32)

    sec = tidx_ref[0, 0, :] // SLICE_RANGE
    onehot = (sec[:, None] == lax.broadcasted_iota(
        jnp.int32, (TBLK, NSEC_PAD), 1)).astype(jnp.bfloat16)
    cc = jnp.dot(onehot, centers_scr[...],
                 preferred_element_type=jnp.float32).astype(jnp.bfloat16)
    diff = temb_ref[...].astype(jnp.bfloat16) - cc
    sq = diff * diff
    ones = jnp.ones((NSEC_PAD, 8), jnp.bfloat16)
    dist2 = jnp.dot(sq, ones, preferred_element_type=jnp.float32)
    dist = jnp.sqrt(dist2[:, 0])
    out_ref[...] += (jnp.sum(dist) * (1.0 / B)).reshape(1, 1)


def _dist_call(sums, tcsum, cnts, temb, tidx_r):
    return pl.pallas_call(
        _dist_body,
        grid=(GRID,),
        in_specs=[
            pl.BlockSpec((NC, NSEC_PAD, D), lambda i: (0, 0, 0)),
            pl.BlockSpec((NSEC_PAD, D), lambda i: (0, 0)),
            pl.BlockSpec((NSEC_PAD, 8), lambda i: (0, 0)),
            pl.BlockSpec((TBLK, D), lambda i: (i, 0)),
            pl.BlockSpec((1, 1, TBLK), lambda i: (i, 0, 0)),
        ],
        out_specs=pl.BlockSpec((1, 1), lambda i: (0, 0)),
        out_shape=jax.ShapeDtypeStruct((1, 1), jnp.float32),
        scratch_shapes=[pltpu.VMEM((NSEC_PAD, D), jnp.bfloat16)],
    )(sums, tcsum, cnts, temb, tidx_r)


def kernel(target_embeddings, target_slice_idx, source_embeddings,
           source_slice_idx, source_sectors):
    del source_slice_idx
    sec32 = source_sectors.astype(jnp.int32)
    zeros_acc = jnp.zeros((NSEC_PAD, D), jnp.float32)

    sums = _seg_call(source_embeddings, sec32, zeros_acc)

    sec_pad = jnp.concatenate(
        [sec32, jnp.full((TSEG_GRID * TSEG - N_SRC,), NSEC_PAD - 1,
                         jnp.int32)])
    tcsum, cnts = _tcseg_call(source_embeddings,
                              sec_pad.reshape(TSEG_GRID, 1, TSEG))

    tidx_r = target_slice_idx.astype(jnp.int32).reshape(GRID, 1, TBLK)
    out = _dist_call(sums, tcsum, cnts, target_embeddings, tidx_r)
    return out[0, 0]
